# baseline (speedup 1.0000x reference)
import functools

import jax
import jax.numpy as jnp
import numpy as np
from jax import lax
from jax.experimental import pallas as pl
from jax.experimental.pallas import tpu as pltpu
from jax.experimental.pallas import tpu_sc as plsc

_DELTA_V = 0.5
_DELTA_D = 3.0
_NL = 4
_R = 8
_NS = 9


def _pick_chunk(hw):
    for c in (131072, 65536, 32768, 16384, 8192, 4096, 2048, 1024, 512,
              256, 128):
        if hw % c == 0:
            return c
    return hw


def _sc_stats(emb3, lab2):
    nb, c, hw = emb3.shape
    info = plsc.get_sparse_core_info()
    nc, ns, lanes = info.num_cores, info.num_subcores, info.num_lanes
    nw = nc * ns
    per_w = hw // nw
    ch = 2048
    while per_w % ch:
        ch //= 2
    nch = per_w // ch
    nstats = _NL * _NS
    mesh = plsc.VectorSubcoreMesh(core_axis_name="c", subcore_axis_name="s")

    @functools.partial(
        pl.kernel,
        mesh=mesh,
        out_type=jax.ShapeDtypeStruct((nb, nstats, nw * lanes), jnp.float32),
        scratch_types=[
            pltpu.VMEM((c, ch), jnp.float32),
            pltpu.VMEM((ch,), jnp.int32),
            pltpu.VMEM((nstats * lanes,), jnp.float32),
            pltpu.SemaphoreType.DMA,
        ],
    )
    def sc_kernel(emb_hbm, lab_hbm, out_hbm, ebuf, lbuf, stage, sem):
        wid = lax.axis_index("s") * nc + lax.axis_index("c")
        base = wid * per_w
        zero = jnp.zeros((lanes,), dtype=jnp.float32)

        for b in range(nb):
            def chunk_body(g, acc):
                off = base + g * ch
                copies = [
                    pltpu.async_copy(
                        emb_hbm.at[b, cc, pl.ds(off, ch)], ebuf.at[cc], sem)
                    for cc in range(c)
                ]
                copies.append(
                    pltpu.async_copy(lab_hbm.at[b, pl.ds(off, ch)], lbuf,
                                     sem))
                for cp in copies:
                    cp.wait()

                def vec_body(i, a):
                    a = list(a)
                    lab_v = lbuf[pl.ds(i * lanes, lanes)]
                    vs = [ebuf[cc, pl.ds(i * lanes, lanes)]
                          for cc in range(c)]
                    one = jnp.ones((lanes,), dtype=jnp.float32)
                    zv = jnp.zeros((lanes,), dtype=jnp.float32)
                    for l in range(_NL):
                        m = lab_v == (l + 1)
                        a[l * _NS + c] = a[l * _NS + c] + jnp.where(m, one,
                                                                    zv)
                        for cc in range(c):
                            a[l * _NS + cc] = a[l * _NS + cc] + jnp.where(
                                m, vs[cc], zv)
                    return tuple(a)

                return lax.fori_loop(0, ch // lanes, vec_body, acc)

            acc = lax.fori_loop(0, nch, chunk_body,
                                tuple(zero for _ in range(nstats)))
            for k in range(nstats):
                stage[pl.ds(k * lanes, lanes)] = acc[k]
            outs = [
                pltpu.async_copy(
                    stage.at[pl.ds(k * lanes, lanes)],
                    out_hbm.at[b, k, pl.ds(wid * lanes, lanes)], sem)
                for k in range(nstats)
            ]
            for cp in outs:
                cp.wait()

    return sc_kernel(emb3, lab2)


def _tc_body(emb_ref, lab_ref, raw_ref, sel_ref, var_ref, dist_ref,
             stats_ref, accb_ref, acc_ref, *, nb, nchunks):
    b = pl.program_id(0)
    j = pl.program_id(1)
    c = emb_ref.shape[1]
    chunk = emb_ref.shape[2]

    @pl.when(j == 0)
    def _():
        accb_ref[...] = jnp.zeros_like(accb_ref)
        raw_b = raw_ref[b]
        raw_sums = lax.dot_general(
            raw_b, jnp.ones((1, raw_b.shape[1]), jnp.float32),
            (((1,), (1,)), ((), ())),
            preferred_element_type=jnp.float32)
        cols = []
        for cc in range(_NS):
            sel = sel_ref[pl.ds(cc * _R, _R), :]
            cols.append(lax.dot_general(
                sel[:, 0:raw_b.shape[0]], raw_sums,
                (((1,), (0,)), ((), ())),
                preferred_element_type=jnp.float32))
        pad = jnp.zeros((_R, 128 - _NS), dtype=jnp.float32)
        stats_ref[pl.ds(b * _R, _R), :] = jnp.concatenate(cols + [pad],
                                                          axis=1)

    emb = emb_ref[0]
    lab = lab_ref[0]
    stats_b = stats_ref[pl.ds(b * _R, _R), :]
    cnt = stats_b[:, c:c + 1]
    safe_cnt = jnp.where(cnt > 0, cnt, 1.0)
    mu = stats_b[:, 0:c] / safe_cnt
    sq_mu = jnp.sum(mu * mu, axis=1, keepdims=True)

    lane_ids = lax.broadcasted_iota(jnp.int32, (_R, chunk), 0) + 1
    masks = (jnp.broadcast_to(lab, (_R, chunk)) == lane_ids).astype(
        jnp.float32)
    ones_sq = jnp.ones((_R, c), dtype=jnp.float32)
    sq_e = lax.dot_general(ones_sq, emb * emb, (((1,), (0,)), ((), ())),
                           preferred_element_type=jnp.float32)
    dots = lax.dot_general(-2.0 * mu, emb, (((1,), (0,)), ((), ())),
                           preferred_element_type=jnp.float32)
    d2 = jnp.maximum(sq_e + dots + sq_mu, 0.0)
    d = jnp.sqrt(d2)
    hinge = jnp.maximum(d - _DELTA_V, 0.0)
    accb_ref[...] += hinge * hinge * masks

    @pl.when(j == nchunks - 1)
    def _():
        lane_sums = jnp.sum(accb_ref[...], axis=1, keepdims=True)
        pad = jnp.zeros((_R, 127), dtype=jnp.float32)
        acc_ref[pl.ds(b * _R, _R), :] = jnp.concatenate(
            [lane_sums, pad], axis=1)

    @pl.when(jnp.logical_and(b == nb - 1, j == nchunks - 1))
    def _():
        var_loss = jnp.float32(0.0)
        dist_loss = jnp.float32(0.0)
        for bb in range(nb):
            stats_bb = stats_ref[bb * _R:(bb + 1) * _R, :]
            cnt_b = stats_bb[0:_NL, c:c + 1]
            has = cnt_b > 0
            safe = jnp.where(has, cnt_b, 1.0)
            varsums = acc_ref[bb * _R:bb * _R + _NL, 0:1]
            batch_var = jnp.sum(jnp.where(has, varsums / safe, 0.0))
            nl = jnp.sum(has.astype(jnp.float32))
            mu_b = jnp.where(has, stats_bb[0:_NL, 0:c] / safe, 0.0)
            contrib = jnp.float32(0.0)
            for i in range(_NL):
                for k in range(i + 1, _NL):
                    diff = mu_b[i:i + 1, :] - mu_b[k:k + 1, :]
                    pd2 = jnp.sum(diff * diff)
                    pd = jnp.where(pd2 > 0,
                                   jnp.sqrt(jnp.where(pd2 > 0, pd2, 1.0)),
                                   0.0)
                    both = (cnt_b[i, 0] * cnt_b[k, 0]) > 0
                    h = jnp.maximum(_DELTA_D - pd, 0.0)
                    contrib += 2.0 * jnp.where(both, h * h, 0.0)
            new_var = (var_loss + batch_var) / nl
            var_loss = jnp.where(nl > 0, new_var, var_loss)
            new_dist = (dist_loss + jnp.where(nl > 1, contrib, 0.0)) / (
                2.0 * nl * (nl - 1.0))
            dist_loss = jnp.where(nl > 0, new_dist, dist_loss)
        var_ref[...] = jnp.reshape(var_loss / nb, (1, 1))
        dist_ref[...] = jnp.reshape(dist_loss / nb, (1, 1))


def _tc_var(emb3, lab3, raw, interpret=False):
    nb, c, hw = emb3.shape
    nwl = raw.shape[2]
    chunk = _pick_chunk(hw)
    nchunks = hw // chunk
    grid = (nb, nchunks)

    sel = np.zeros((_NS * _R, 128), dtype=np.float32)
    for cc in range(_NS):
        for l in range(_NL):
            sel[cc * _R + l, l * _NS + cc] = 1.0
    sel = jnp.asarray(sel)

    var, dist = pl.pallas_call(
        functools.partial(_tc_body, nb=nb, nchunks=nchunks),
        grid=grid,
        in_specs=[pl.BlockSpec((1, c, chunk), lambda b, j: (b, 0, j)),
                  pl.BlockSpec((1, 1, chunk), lambda b, j: (b, 0, j)),
                  pl.BlockSpec((nb, _NL * _NS, nwl),
                               lambda b, j: (0, 0, 0)),
                  pl.BlockSpec((_NS * _R, 128), lambda b, j: (0, 0))],
        out_specs=[pl.BlockSpec((1, 1), lambda b, j: (0, 0)),
                   pl.BlockSpec((1, 1), lambda b, j: (0, 0))],
        out_shape=[jax.ShapeDtypeStruct((1, 1), jnp.float32),
                   jax.ShapeDtypeStruct((1, 1), jnp.float32)],
        scratch_shapes=[pltpu.VMEM((_R * nb, 128), jnp.float32),
                        pltpu.VMEM((_R, chunk), jnp.float32),
                        pltpu.VMEM((_R * nb, 128), jnp.float32)],
        compiler_params=pltpu.CompilerParams(
            dimension_semantics=("arbitrary", "arbitrary")),
        interpret=interpret,
    )(emb3, lab3, raw, sel)

    return var[0, 0], dist[0, 0]


def kernel(embedding_tensor, instance_labels):
    nb, c, h, w = embedding_tensor.shape
    hw = h * w
    emb3 = embedding_tensor.reshape(nb, c, hw)
    lab2 = instance_labels.reshape(nb, hw).astype(jnp.int32)
    lab3 = lab2.reshape(nb, 1, hw)
    raw = _sc_stats(emb3, lab2)
    return _tc_var(emb3, lab3, raw)


# speedup vs baseline: 1.0120x; 1.0120x over previous
import functools

import jax
import jax.numpy as jnp
import numpy as np
from jax import lax
from jax.experimental import pallas as pl
from jax.experimental.pallas import tpu as pltpu
from jax.experimental.pallas import tpu_sc as plsc

_DELTA_V = 0.5
_DELTA_D = 3.0
_NL = 4
_R = 8
_NS = 9


def _pick_chunk(hw):
    for c in (131072, 65536, 32768, 16384, 8192, 4096, 2048, 1024, 512,
              256, 128):
        if hw % c == 0:
            return c
    return hw


def _sc_stats(emb3, lab2):
    nb, c, hw = emb3.shape
    info = plsc.get_sparse_core_info()
    nc, ns, lanes = info.num_cores, info.num_subcores, info.num_lanes
    nw = nc * ns
    per_w = hw // nw
    ch = 2048
    while per_w % ch:
        ch //= 2
    nch = per_w // ch
    nstats = _NL * _NS
    mesh = plsc.VectorSubcoreMesh(core_axis_name="c", subcore_axis_name="s")

    two_slot = nch % 2 == 0 and nch >= 2

    @functools.partial(
        pl.kernel,
        mesh=mesh,
        out_type=jax.ShapeDtypeStruct((nb, nstats, nw * lanes), jnp.float32),
        scratch_types=[
            pltpu.VMEM((2, c, ch), jnp.float32),
            pltpu.VMEM((2, ch), jnp.int32),
            pltpu.VMEM((nstats * lanes,), jnp.float32),
            pltpu.SemaphoreType.DMA,
            pltpu.SemaphoreType.DMA,
        ],
    )
    def sc_kernel(emb_hbm, lab_hbm, out_hbm, ebuf, lbuf, stage, sem0, sem1):
        wid = lax.axis_index("s") * nc + lax.axis_index("c")
        base = wid * per_w
        zero = jnp.zeros((lanes,), dtype=jnp.float32)
        one = jnp.ones((lanes,), dtype=jnp.float32)
        zv = jnp.zeros((lanes,), dtype=jnp.float32)
        sems = [sem0, sem1]

        def copies(b, g, s):
            off = base + g * ch
            cps = [
                pltpu.make_async_copy(
                    emb_hbm.at[b, cc, pl.ds(off, ch)], ebuf.at[s, cc],
                    sems[s])
                for cc in range(c)
            ]
            cps.append(
                pltpu.make_async_copy(lab_hbm.at[b, pl.ds(off, ch)],
                                      lbuf.at[s], sems[s]))
            return cps

        def fire(b, g, s):
            for cp in copies(b, g, s):
                cp.start()

        def drain(b, g, s):
            for cp in copies(b, g, s):
                cp.wait()

        def consume(s, acc):
            def vec_body(i, a):
                a = list(a)
                lab_v = lbuf[s, pl.ds(i * lanes, lanes)]
                vs = [ebuf[s, cc, pl.ds(i * lanes, lanes)]
                      for cc in range(c)]
                for l in range(_NL):
                    m = jnp.where(lab_v == (l + 1), one, zv)
                    a[l * _NS + c] = a[l * _NS + c] + m
                    for cc in range(c):
                        a[l * _NS + cc] = a[l * _NS + cc] + vs[cc] * m
                return tuple(a)

            return lax.fori_loop(0, ch // lanes, vec_body, acc, unroll=2)

        for b in range(nb):
            acc = tuple(zero for _ in range(nstats))
            if two_slot:
                fire(b, 0, 0)

                def pair_body(g2, a):
                    g = g2 * 2
                    fire(b, g + 1, 1)
                    drain(b, g, 0)
                    a = consume(0, a)

                    @pl.when(g2 + 1 < nch // 2)
                    def _():
                        fire(b, g + 2, 0)

                    drain(b, g + 1, 1)
                    return consume(1, a)

                acc = lax.fori_loop(0, nch // 2, pair_body, acc)
            else:
                def chunk_body(g, a):
                    fire(b, g, 0)
                    drain(b, g, 0)
                    return consume(0, a)

                acc = lax.fori_loop(0, nch, chunk_body, acc)

            for k in range(nstats):
                stage[pl.ds(k * lanes, lanes)] = acc[k]
            outs = [
                pltpu.make_async_copy(
                    stage.at[pl.ds(k * lanes, lanes)],
                    out_hbm.at[b, k, pl.ds(wid * lanes, lanes)], sem0)
                for k in range(nstats)
            ]
            for cp in outs:
                cp.start()
            for cp in outs:
                cp.wait()

    return sc_kernel(emb3, lab2)


def _tc_body(emb_ref, lab_ref, raw_ref, sel_ref, var_ref, dist_ref,
             stats_ref, accb_ref, acc_ref, *, nb, nchunks):
    b = pl.program_id(0)
    j = pl.program_id(1)
    c = emb_ref.shape[1]
    chunk = emb_ref.shape[2]

    @pl.when(j == 0)
    def _():
        accb_ref[...] = jnp.zeros_like(accb_ref)
        raw_b = raw_ref[b]
        raw_sums = lax.dot_general(
            raw_b, jnp.ones((1, raw_b.shape[1]), jnp.float32),
            (((1,), (1,)), ((), ())),
            preferred_element_type=jnp.float32)
        cols = []
        for cc in range(_NS):
            sel = sel_ref[pl.ds(cc * _R, _R), :]
            cols.append(lax.dot_general(
                sel[:, 0:raw_b.shape[0]], raw_sums,
                (((1,), (0,)), ((), ())),
                preferred_element_type=jnp.float32))
        pad = jnp.zeros((_R, 128 - _NS), dtype=jnp.float32)
        stats_ref[pl.ds(b * _R, _R), :] = jnp.concatenate(cols + [pad],
                                                          axis=1)

    emb = emb_ref[0]
    lab = lab_ref[0]
    stats_b = stats_ref[pl.ds(b * _R, _R), :]
    cnt = stats_b[:, c:c + 1]
    safe_cnt = jnp.where(cnt > 0, cnt, 1.0)
    mu = stats_b[:, 0:c] / safe_cnt
    sq_mu = jnp.sum(mu * mu, axis=1, keepdims=True)

    lane_ids = lax.broadcasted_iota(jnp.int32, (_R, chunk), 0) + 1
    masks = (jnp.broadcast_to(lab, (_R, chunk)) == lane_ids).astype(
        jnp.float32)
    ones_sq = jnp.ones((_R, c), dtype=jnp.float32)
    sq_e = lax.dot_general(ones_sq, emb * emb, (((1,), (0,)), ((), ())),
                           preferred_element_type=jnp.float32)
    dots = lax.dot_general(-2.0 * mu, emb, (((1,), (0,)), ((), ())),
                           preferred_element_type=jnp.float32)
    d2 = jnp.maximum(sq_e + dots + sq_mu, 0.0)
    d = jnp.sqrt(d2)
    hinge = jnp.maximum(d - _DELTA_V, 0.0)
    accb_ref[...] += hinge * hinge * masks

    @pl.when(j == nchunks - 1)
    def _():
        lane_sums = jnp.sum(accb_ref[...], axis=1, keepdims=True)
        pad = jnp.zeros((_R, 127), dtype=jnp.float32)
        acc_ref[pl.ds(b * _R, _R), :] = jnp.concatenate(
            [lane_sums, pad], axis=1)

    @pl.when(jnp.logical_and(b == nb - 1, j == nchunks - 1))
    def _():
        var_loss = jnp.float32(0.0)
        dist_loss = jnp.float32(0.0)
        for bb in range(nb):
            stats_bb = stats_ref[bb * _R:(bb + 1) * _R, :]
            cnt_b = stats_bb[0:_NL, c:c + 1]
            has = cnt_b > 0
            safe = jnp.where(has, cnt_b, 1.0)
            varsums = acc_ref[bb * _R:bb * _R + _NL, 0:1]
            batch_var = jnp.sum(jnp.where(has, varsums / safe, 0.0))
            nl = jnp.sum(has.astype(jnp.float32))
            mu_b = jnp.where(has, stats_bb[0:_NL, 0:c] / safe, 0.0)
            contrib = jnp.float32(0.0)
            for i in range(_NL):
                for k in range(i + 1, _NL):
                    diff = mu_b[i:i + 1, :] - mu_b[k:k + 1, :]
                    pd2 = jnp.sum(diff * diff)
                    pd = jnp.where(pd2 > 0,
                                   jnp.sqrt(jnp.where(pd2 > 0, pd2, 1.0)),
                                   0.0)
                    both = (cnt_b[i, 0] * cnt_b[k, 0]) > 0
                    h = jnp.maximum(_DELTA_D - pd, 0.0)
                    contrib += 2.0 * jnp.where(both, h * h, 0.0)
            new_var = (var_loss + batch_var) / nl
            var_loss = jnp.where(nl > 0, new_var, var_loss)
            new_dist = (dist_loss + jnp.where(nl > 1, contrib, 0.0)) / (
                2.0 * nl * (nl - 1.0))
            dist_loss = jnp.where(nl > 0, new_dist, dist_loss)
        var_ref[...] = jnp.reshape(var_loss / nb, (1, 1))
        dist_ref[...] = jnp.reshape(dist_loss / nb, (1, 1))


def _tc_var(emb3, lab3, raw, interpret=False):
    nb, c, hw = emb3.shape
    nwl = raw.shape[2]
    chunk = _pick_chunk(hw)
    nchunks = hw // chunk
    grid = (nb, nchunks)

    sel = np.zeros((_NS * _R, 128), dtype=np.float32)
    for cc in range(_NS):
        for l in range(_NL):
            sel[cc * _R + l, l * _NS + cc] = 1.0
    sel = jnp.asarray(sel)

    var, dist = pl.pallas_call(
        functools.partial(_tc_body, nb=nb, nchunks=nchunks),
        grid=grid,
        in_specs=[pl.BlockSpec((1, c, chunk), lambda b, j: (b, 0, j)),
                  pl.BlockSpec((1, 1, chunk), lambda b, j: (b, 0, j)),
                  pl.BlockSpec((nb, _NL * _NS, nwl),
                               lambda b, j: (0, 0, 0)),
                  pl.BlockSpec((_NS * _R, 128), lambda b, j: (0, 0))],
        out_specs=[pl.BlockSpec((1, 1), lambda b, j: (0, 0)),
                   pl.BlockSpec((1, 1), lambda b, j: (0, 0))],
        out_shape=[jax.ShapeDtypeStruct((1, 1), jnp.float32),
                   jax.ShapeDtypeStruct((1, 1), jnp.float32)],
        scratch_shapes=[pltpu.VMEM((_R * nb, 128), jnp.float32),
                        pltpu.VMEM((_R, chunk), jnp.float32),
                        pltpu.VMEM((_R * nb, 128), jnp.float32)],
        compiler_params=pltpu.CompilerParams(
            dimension_semantics=("arbitrary", "arbitrary")),
        interpret=interpret,
    )(emb3, lab3, raw, sel)

    return var[0, 0], dist[0, 0]


def kernel(embedding_tensor, instance_labels):
    nb, c, h, w = embedding_tensor.shape
    hw = h * w
    emb3 = embedding_tensor.reshape(nb, c, hw)
    lab2 = instance_labels.reshape(nb, hw).astype(jnp.int32)
    lab3 = lab2.reshape(nb, 1, hw)
    raw = _sc_stats(emb3, lab2)
    return _tc_var(emb3, lab3, raw)
